# Initial kernel scaffold; baseline (speedup 1.0000x reference)
import jax
import jax.numpy as jnp
from jax.experimental import pallas as pl


def kernel(inputs_ids, tok_table, pos_table):
    raise NotImplementedError("write your pallas kernel here")


# speedup vs baseline: 1.2731x; 1.2731x over previous
import functools

import jax
import jax.numpy as jnp
from jax import lax
from jax.experimental import pallas as pl
from jax.experimental.pallas import tpu as pltpu
from jax.experimental.pallas import tpu_sc as plsc

VOCAB = 100000
MAX_LEN = 2048
EMB = 128
B, L = 4, 2048
N_ROWS = B * L

_info = plsc.get_sparse_core_info()
NC, NS = _info.num_cores, _info.num_subcores
NW = NC * NS
ROWS_PER_W = N_ROWS // NW
IDX_CHUNK = 128
N_CHUNKS = ROWS_PER_W // IDX_CHUNK


def _body(ids_hbm, tok_hbm, pos_hbm, out_hbm, idx_v, rows_v, pos_v, sem):
    wid = lax.axis_index("s") * NC + lax.axis_index("c")
    base = wid * ROWS_PER_W
    pos_base = lax.rem(base, MAX_LEN)

    pltpu.sync_copy(ids_hbm.at[pl.ds(wid * N_CHUNKS, N_CHUNKS)], idx_v)

    copies = []
    for j in range(N_CHUNKS):
        copies.append(
            pltpu.async_copy(
                tok_hbm.at[idx_v.at[j]],
                rows_v.at[pl.ds(j * IDX_CHUNK, IDX_CHUNK)],
                sem,
            )
        )
    pltpu.sync_copy(pos_hbm.at[pl.ds(pos_base, ROWS_PER_W)], pos_v)
    for cp in copies:
        cp.wait()

    def add_row(i, _):
        for j in range(EMB // 16):
            s = pl.ds(j * 16, 16)
            rows_v[i, s] = rows_v[i, s] + pos_v[i, s]
        return 0

    lax.fori_loop(0, ROWS_PER_W, add_row, 0, unroll=False)

    pltpu.sync_copy(rows_v, out_hbm.at[pl.ds(base, ROWS_PER_W)])


@jax.jit
def _embed(ids2d, tok_table, pos_table):
    mesh = plsc.VectorSubcoreMesh(core_axis_name="c", subcore_axis_name="s")
    k = functools.partial(
        pl.kernel,
        mesh=mesh,
        out_type=jax.ShapeDtypeStruct((N_ROWS, EMB), jnp.float32),
        scratch_types=[
            pltpu.VMEM((N_CHUNKS, IDX_CHUNK), jnp.int32),
            pltpu.VMEM((ROWS_PER_W, EMB), jnp.float32),
            pltpu.VMEM((ROWS_PER_W, EMB), jnp.float32),
            pltpu.SemaphoreType.DMA,
        ],
    )(_body)
    return k(ids2d, tok_table, pos_table)


def kernel(inputs_ids, tok_table, pos_table):
    ids2d = inputs_ids.reshape(N_ROWS // IDX_CHUNK, IDX_CHUNK)
    out = _embed(ids2d, tok_table, pos_table)
    return out.reshape(B, L, EMB)


# speedup vs baseline: 1.3347x; 1.0484x over previous
import functools

import jax
import jax.numpy as jnp
from jax import lax
from jax.experimental import pallas as pl
from jax.experimental.pallas import tpu as pltpu
from jax.experimental.pallas import tpu_sc as plsc

VOCAB = 100000
MAX_LEN = 2048
EMB = 128
B, L = 4, 2048
N_ROWS = B * L

_info = plsc.get_sparse_core_info()
NC, NS = _info.num_cores, _info.num_subcores
NW = NC * NS
ROWS_PER_W = N_ROWS // NW
IDX_CHUNK = 128
N_CHUNKS = ROWS_PER_W // IDX_CHUNK


def _body(ids_hbm, tok_hbm, pos_hbm, out_hbm, idx_v, rows_v, sem):
    wid = lax.axis_index("s") * NC + lax.axis_index("c")
    base = wid * ROWS_PER_W
    pos_base = lax.rem(base, MAX_LEN)

    pltpu.sync_copy(ids_hbm.at[pl.ds(wid * N_CHUNKS, N_CHUNKS)], idx_v)

    pltpu.sync_copy(pos_hbm.at[pl.ds(pos_base, ROWS_PER_W)], rows_v)
    copies = []
    for j in range(N_CHUNKS):
        copies.append(
            pltpu.async_copy(
                tok_hbm.at[idx_v.at[j]],
                rows_v.at[pl.ds(j * IDX_CHUNK, IDX_CHUNK)],
                sem,
                add=True,
            )
        )
    for cp in copies:
        cp.wait()

    pltpu.sync_copy(rows_v, out_hbm.at[pl.ds(base, ROWS_PER_W)])


@jax.jit
def _embed(ids2d, tok_table, pos_table):
    mesh = plsc.VectorSubcoreMesh(core_axis_name="c", subcore_axis_name="s")
    k = functools.partial(
        pl.kernel,
        mesh=mesh,
        out_type=jax.ShapeDtypeStruct((N_ROWS, EMB), jnp.float32),
        scratch_types=[
            pltpu.VMEM((N_CHUNKS, IDX_CHUNK), jnp.int32),
            pltpu.VMEM((ROWS_PER_W, EMB), jnp.float32),
            pltpu.SemaphoreType.DMA,
        ],
    )(_body)
    return k(ids2d, tok_table, pos_table)


def kernel(inputs_ids, tok_table, pos_table):
    ids2d = inputs_ids.reshape(N_ROWS // IDX_CHUNK, IDX_CHUNK)
    out = _embed(ids2d, tok_table, pos_table)
    return out.reshape(B, L, EMB)


# speedup vs baseline: 1.3775x; 1.0320x over previous
import functools

import jax
import jax.numpy as jnp
from jax import lax
from jax.experimental import pallas as pl
from jax.experimental.pallas import tpu as pltpu
from jax.experimental.pallas import tpu_sc as plsc

VOCAB = 100000
MAX_LEN = 2048
EMB = 128
B, L = 4, 2048
N_ROWS = B * L

_info = plsc.get_sparse_core_info()
NC, NS = _info.num_cores, _info.num_subcores
NW = NC * NS
ROWS_PER_W = N_ROWS // NW
CHUNK = 64
N_CH = ROWS_PER_W // CHUNK


def _body(ids_hbm, tok_hbm, pos_hbm, out_hbm, idx_v, rows_v,
          sem_i, sem_p, sem_g, sem_o):
    wid = lax.axis_index("s") * NC + lax.axis_index("c")
    base = wid * ROWS_PER_W
    pos_base = lax.rem(base, MAX_LEN)

    idx_cp = pltpu.async_copy(
        ids_hbm.at[pl.ds(wid * N_CH, N_CH)], idx_v, sem_i)

    pos_cps = []
    for c in range(N_CH):
        pos_cps.append(pltpu.async_copy(
            pos_hbm.at[pl.ds(pos_base + c * CHUNK, CHUNK)],
            rows_v.at[pl.ds(c * CHUNK, CHUNK)],
            sem_p.at[c]))
    idx_cp.wait()

    g_cps = []
    for c in range(N_CH):
        pos_cps[c].wait()
        g_cps.append(pltpu.async_copy(
            tok_hbm.at[idx_v.at[c]],
            rows_v.at[pl.ds(c * CHUNK, CHUNK)],
            sem_g.at[c],
            add=True))

    o_cps = []
    for c in range(N_CH):
        g_cps[c].wait()
        o_cps.append(pltpu.async_copy(
            rows_v.at[pl.ds(c * CHUNK, CHUNK)],
            out_hbm.at[pl.ds(base + c * CHUNK, CHUNK)],
            sem_o.at[c]))
    for cp in o_cps:
        cp.wait()


@jax.jit
def _embed(ids2d, tok_table, pos_table):
    mesh = plsc.VectorSubcoreMesh(core_axis_name="c", subcore_axis_name="s")
    k = functools.partial(
        pl.kernel,
        mesh=mesh,
        out_type=jax.ShapeDtypeStruct((N_ROWS, EMB), jnp.float32),
        scratch_types=[
            pltpu.VMEM((N_CH, CHUNK), jnp.int32),
            pltpu.VMEM((ROWS_PER_W, EMB), jnp.float32),
            pltpu.SemaphoreType.DMA,
            pltpu.SemaphoreType.DMA((N_CH,)),
            pltpu.SemaphoreType.DMA((N_CH,)),
            pltpu.SemaphoreType.DMA((N_CH,)),
        ],
    )(_body)
    return k(ids2d, tok_table, pos_table)


def kernel(inputs_ids, tok_table, pos_table):
    ids2d = inputs_ids.reshape(N_ROWS // CHUNK, CHUNK)
    out = _embed(ids2d, tok_table, pos_table)
    return out.reshape(B, L, EMB)
